# baseline (speedup 1.0000x reference)
import functools

import jax
import jax.numpy as jnp
import numpy as np
from jax import lax
from jax.experimental import pallas as pl
from jax.experimental.pallas import tpu as pltpu
from jax.experimental.pallas import tpu_sc as plsc

B = 4096
F = 26
V = 100000
D = 64
DIN = 13
NF = F + 1
NN = NF * NF
BB = 512


def _sc_gather(table, idx):
    info = plsc.get_sparse_core_info()
    nc, ns = info.num_cores, info.num_subcores
    nw = nc * ns
    n = idx.shape[0]
    per_w = n // nw
    ch = 128
    n_ch = per_w // ch
    assert per_w % ch == 0

    mesh = plsc.VectorSubcoreMesh(core_axis_name="c", subcore_axis_name="s")

    @functools.partial(
        pl.kernel,
        mesh=mesh,
        out_type=jax.ShapeDtypeStruct((n, D), jnp.float32),
        scratch_types=[
            pltpu.VMEM((ch,), jnp.int32),
            pltpu.VMEM((ch, D), jnp.float32),
            pltpu.SemaphoreType.DMA,
        ],
        compiler_params=pltpu.CompilerParams(use_tc_tiling_on_sc=False),
    )
    def k(table_hbm, idx_hbm, out_hbm, idx_v, rows_v, sem):
        wid = lax.axis_index("s") * nc + lax.axis_index("c")
        base = wid * per_w

        def body(i, carry):
            off = base + i * ch
            pltpu.sync_copy(idx_hbm.at[pl.ds(off, ch)], idx_v)
            pltpu.async_copy(table_hbm.at[idx_v], rows_v, sem).wait()
            pltpu.sync_copy(rows_v, out_hbm.at[pl.ds(off, ch)])
            return carry

        lax.fori_loop(0, n_ch, body, 0)

    return k(table, idx)


def _tc_body(d_ref, emb_ref, dw0, db0, dw1, db1, dw2, db2,
             ow0a, w729, ob0, ow1, ob1, ow2, ob2, ow3, ob3, ow4, ob4,
             out_ref):
    f32 = jnp.float32
    x = d_ref[...]
    x = jnp.maximum(jnp.dot(x, dw0[...], preferred_element_type=f32) + db0[...], 0.0)
    x = jnp.maximum(jnp.dot(x, dw1[...], preferred_element_type=f32) + db1[...], 0.0)
    x = jnp.maximum(jnp.dot(x, dw2[...], preferred_element_type=f32) + db2[...], 0.0)
    emb = emb_ref[...]
    comb = jnp.concatenate([x[:, None, :], emb], axis=1)
    z = lax.dot_general(comb, comb, (((2,), (2,)), ((0,), (0,))),
                        preferred_element_type=f32)
    zf = z.reshape(BB, NN)
    h = (jnp.dot(x, ow0a[...], preferred_element_type=f32)
         + jnp.dot(zf, w729[...], preferred_element_type=f32) + ob0[...])
    h = jnp.maximum(h, 0.0)
    h = jnp.maximum(jnp.dot(h, ow1[...], preferred_element_type=f32) + ob1[...], 0.0)
    h = jnp.maximum(jnp.dot(h, ow2[...], preferred_element_type=f32) + ob2[...], 0.0)
    h = jnp.maximum(jnp.dot(h, ow3[...], preferred_element_type=f32) + ob3[...], 0.0)
    out_ref[...] = jnp.dot(h, ow4[...], preferred_element_type=f32) + ob4[...]


_IU, _JU = np.triu_indices(NF, k=1)
_PAIR_FLAT = (_IU * NF + _JU).astype(np.int32)


def _tc_forward(dense_features, emb, dw0, db0, dw1, db1, dw2, db2,
                ow0, ob0, ow1, ob1, ow2, ob2, ow3, ob3, ow4, ob4):
    ow0a = ow0[:D]
    w729 = jnp.zeros((NN, ow0.shape[1]), jnp.float32).at[_PAIR_FLAT].set(ow0[D:])

    def row(b):
        return b.reshape(1, -1)

    grid = B // BB
    weights = (dw0, row(db0), dw1, row(db1), dw2, row(db2),
               ow0a, w729, row(ob0), ow1, row(ob1), ow2, row(ob2),
               ow3, row(ob3), ow4, row(ob4))

    def wspec(w):
        return pl.BlockSpec(w.shape, lambda i: (0,) * w.ndim)

    out = pl.pallas_call(
        _tc_body,
        grid=(grid,),
        in_specs=[
            pl.BlockSpec((BB, DIN), lambda i: (i, 0)),
            pl.BlockSpec((BB, F, D), lambda i: (i, 0, 0)),
        ] + [wspec(w) for w in weights],
        out_specs=pl.BlockSpec((BB, 1), lambda i: (i, 0)),
        out_shape=jax.ShapeDtypeStruct((B, 1), jnp.float32),
    )(dense_features, emb, *weights)
    return out


def kernel(dense_features, sparse_features, emb_tables,
           dw0, db0, dw1, db1, dw2, db2,
           ow0, ob0, ow1, ob1, ow2, ob2, ow3, ob3, ow4, ob4):
    table = emb_tables.reshape(F * V, D)
    idx = (sparse_features.astype(jnp.int32)
           + (jnp.arange(F, dtype=jnp.int32) * V)[None, :]).reshape(B * F)
    emb_flat = _sc_gather(table, idx)
    emb = emb_flat.reshape(B, F, D)
    return _tc_forward(dense_features, emb, dw0, db0, dw1, db1, dw2, db2,
                       ow0, ob0, ow1, ob1, ow2, ob2, ow3, ob3, ow4, ob4)


# speedup vs baseline: 1.3059x; 1.3059x over previous
import functools

import jax
import jax.numpy as jnp
import numpy as np
from jax import lax
from jax.experimental import pallas as pl
from jax.experimental.pallas import tpu as pltpu
from jax.experimental.pallas import tpu_sc as plsc

B = 4096
F = 26
V = 100000
D = 64
DIN = 13
NF = F + 1
NN = NF * NF
BB = 512


def _sc_gather(table, idx):
    info = plsc.get_sparse_core_info()
    nc, ns = info.num_cores, info.num_subcores
    nw = nc * ns
    n = idx.shape[0]
    per_w = n // nw
    ch = 128
    n_ch = per_w // ch
    assert per_w % ch == 0

    mesh = plsc.VectorSubcoreMesh(core_axis_name="c", subcore_axis_name="s")

    @functools.partial(
        pl.kernel,
        mesh=mesh,
        out_type=jax.ShapeDtypeStruct((n, 2 * D), jnp.float32),
        scratch_types=[
            pltpu.VMEM((ch,), jnp.int32),
            pltpu.VMEM((ch, 2 * D), jnp.float32),
            pltpu.SemaphoreType.DMA,
        ],
        compiler_params=pltpu.CompilerParams(use_tc_tiling_on_sc=True),
    )
    def k(table_hbm, idx_hbm, out_hbm, idx_v, rows_v, sem):
        wid = lax.axis_index("s") * nc + lax.axis_index("c")
        base = wid * per_w

        def body(i, carry):
            off = base + i * ch
            pltpu.sync_copy(idx_hbm.at[pl.ds(off, ch)], idx_v)
            pltpu.async_copy(table_hbm.at[idx_v], rows_v, sem).wait()
            pltpu.sync_copy(rows_v, out_hbm.at[pl.ds(off, ch)])
            return carry

        lax.fori_loop(0, n_ch, body, 0)

    return k(table, idx)


def _tc_body(d_ref, emb_ref, dw0, db0, dw1, db1, dw2, db2,
             ow0a, w729, ob0, ow1, ob1, ow2, ob2, ow3, ob3, ow4, ob4,
             out_ref):
    f32 = jnp.float32
    x = d_ref[...]
    x = jnp.maximum(jnp.dot(x, dw0[...], preferred_element_type=f32) + db0[...], 0.0)
    x = jnp.maximum(jnp.dot(x, dw1[...], preferred_element_type=f32) + db1[...], 0.0)
    x = jnp.maximum(jnp.dot(x, dw2[...], preferred_element_type=f32) + db2[...], 0.0)
    pk = emb_ref[...]
    rp = lax.broadcasted_iota(jnp.int32, (BB * F, 1), 0) % 2
    sel = jnp.broadcast_to(rp == 1, (BB * F, D))
    emb2 = jnp.where(sel, pk[:, D:], pk[:, :D])
    emb = emb2.reshape(BB, F, D)
    comb = jnp.concatenate([x[:, None, :], emb], axis=1)
    z = lax.dot_general(comb, comb, (((2,), (2,)), ((0,), (0,))),
                        preferred_element_type=f32)
    zf = z.reshape(BB, NN)
    h = (jnp.dot(x, ow0a[...], preferred_element_type=f32)
         + jnp.dot(zf, w729[...], preferred_element_type=f32) + ob0[...])
    h = jnp.maximum(h, 0.0)
    h = jnp.maximum(jnp.dot(h, ow1[...], preferred_element_type=f32) + ob1[...], 0.0)
    h = jnp.maximum(jnp.dot(h, ow2[...], preferred_element_type=f32) + ob2[...], 0.0)
    h = jnp.maximum(jnp.dot(h, ow3[...], preferred_element_type=f32) + ob3[...], 0.0)
    out_ref[...] = jnp.dot(h, ow4[...], preferred_element_type=f32) + ob4[...]


_IU, _JU = np.triu_indices(NF, k=1)
_PAIR_FLAT = (_IU * NF + _JU).astype(np.int32)


def _tc_forward(dense_features, emb_flat, dw0, db0, dw1, db1, dw2, db2,
                ow0, ob0, ow1, ob1, ow2, ob2, ow3, ob3, ow4, ob4):
    ow0a = ow0[:D]
    w729 = jnp.zeros((NN, ow0.shape[1]), jnp.float32).at[_PAIR_FLAT].set(ow0[D:])

    def row(b):
        return b.reshape(1, -1)

    grid = B // BB
    weights = (dw0, row(db0), dw1, row(db1), dw2, row(db2),
               ow0a, w729, row(ob0), ow1, row(ob1), ow2, row(ob2),
               ow3, row(ob3), ow4, row(ob4))

    def wspec(w):
        return pl.BlockSpec(w.shape, lambda i: (0,) * w.ndim)

    out = pl.pallas_call(
        _tc_body,
        grid=(grid,),
        in_specs=[
            pl.BlockSpec((BB, DIN), lambda i: (i, 0)),
            pl.BlockSpec((BB * F, 2 * D), lambda i: (i, 0)),
        ] + [wspec(w) for w in weights],
        out_specs=pl.BlockSpec((BB, 1), lambda i: (i, 0)),
        out_shape=jax.ShapeDtypeStruct((B, 1), jnp.float32),
    )(dense_features, emb_flat, *weights)
    return out


def kernel(dense_features, sparse_features, emb_tables,
           dw0, db0, dw1, db1, dw2, db2,
           ow0, ob0, ow1, ob1, ow2, ob2, ow3, ob3, ow4, ob4):
    tt = jnp.transpose(emb_tables, (0, 2, 1)).reshape(F * D, V)
    table_pk = jnp.transpose(tt, (1, 0)).reshape(V * (F // 2), 2 * D)
    v = sparse_features.astype(jnp.int32)
    idx = (v * (F // 2)
           + (jnp.arange(F, dtype=jnp.int32) // 2)[None, :]).reshape(B * F)
    emb_flat = _sc_gather(table_pk, idx)
    return _tc_forward(dense_features, emb_flat, dw0, db0, dw1, db1, dw2, db2,
                       ow0, ob0, ow1, ob1, ow2, ob2, ow3, ob3, ow4, ob4)
